# baseline (speedup 1.0000x reference)
import functools

import jax
import jax.numpy as jnp
from jax import lax
from jax.experimental import pallas as pl
from jax.experimental.pallas import tpu as pltpu
from jax.experimental.pallas import tpu_sc as plsc

_TOKENS = 32768
_D_IN = 768
_D_HID = 256
_N_EXPERTS = 64
_BLOCK = 4096

_NW = 32
_TPW = _TOKENS // _NW
_LANES = 16
_CHUNKS = _TPW // _LANES


def _mlp_body(x_ref, w1_ref, b1_ref, w2_ref, b2_ref, logits_ref):
    h = jnp.dot(x_ref[...], w1_ref[...], preferred_element_type=jnp.float32)
    h = jnp.maximum(h + b1_ref[...], 0.0)
    logits = jnp.dot(h, w2_ref[...], preferred_element_type=jnp.float32)
    logits_ref[...] = logits + b2_ref[...]


def _sc_topk_body(logits_hbm, i1_hbm, i2_hbm, g1_hbm, g2_hbm,
                  lv, i1v, i2v, g1v, g2v):
    wid = lax.axis_index("s") * 2 + lax.axis_index("c")
    base = wid * _TPW
    lvf = lv
    pltpu.sync_copy(logits_hbm.at[pl.ds(base * _N_EXPERTS,
                                        _TPW * _N_EXPERTS)], lvf)

    lane = lax.broadcasted_iota(jnp.int32, (_LANES,), 0)

    _G = 4

    def chunk(c, _):
        flat0 = [(c * (_G * _LANES) + g * _LANES + lane) * _N_EXPERTS
                 for g in range(_G)]
        m1 = [plsc.load_gather(lvf, [f]) for f in flat0]
        i1 = [jnp.zeros((_LANES,), jnp.int32) for _g in range(_G)]
        m2 = [jnp.full((_LANES,), -jnp.inf, jnp.float32) for _g in range(_G)]
        i2 = [jnp.zeros((_LANES,), jnp.int32) for _g in range(_G)]
        for e in range(1, _N_EXPERTS):
            for g in range(_G):
                v = plsc.load_gather(lvf, [flat0[g] + e])
                gt1 = v > m1[g]
                gt2 = v > m2[g]
                m2[g] = jnp.where(gt1, m1[g], jnp.where(gt2, v, m2[g]))
                i2[g] = jnp.where(gt1, i1[g], jnp.where(gt2, e, i2[g]))
                m1[g] = jnp.where(gt1, v, m1[g])
                i1[g] = jnp.where(gt1, e, i1[g])
        for g in range(_G):
            ex = jnp.exp(m2[g] - m1[g])
            denom = 1.0 + ex
            sl = pl.ds((c * _G + g) * _LANES, _LANES)
            i1v[sl] = i1[g]
            i2v[sl] = i2[g]
            g1v[sl] = 1.0 / denom
            g2v[sl] = ex / denom
        return _

    lax.fori_loop(0, _CHUNKS // _G, chunk, 0)
    pltpu.sync_copy(i1v, i1_hbm.at[pl.ds(base, _TPW)])
    pltpu.sync_copy(i2v, i2_hbm.at[pl.ds(base, _TPW)])
    pltpu.sync_copy(g1v, g1_hbm.at[pl.ds(base, _TPW)])
    pltpu.sync_copy(g2v, g2_hbm.at[pl.ds(base, _TPW)])


_sc_topk = functools.partial(
    pl.kernel,
    mesh=plsc.VectorSubcoreMesh(core_axis_name="c", subcore_axis_name="s"),
    out_type=[
        jax.ShapeDtypeStruct((_TOKENS,), jnp.int32),
        jax.ShapeDtypeStruct((_TOKENS,), jnp.int32),
        jax.ShapeDtypeStruct((_TOKENS,), jnp.float32),
        jax.ShapeDtypeStruct((_TOKENS,), jnp.float32),
    ],
    scratch_types=[
        pltpu.VMEM((_TPW * _N_EXPERTS,), jnp.float32),
        pltpu.VMEM((_TPW,), jnp.int32),
        pltpu.VMEM((_TPW,), jnp.int32),
        pltpu.VMEM((_TPW,), jnp.float32),
        pltpu.VMEM((_TPW,), jnp.float32),
    ],
    compiler_params=pltpu.CompilerParams(needs_layout_passes=False),
)(_sc_topk_body)


@jax.jit
def kernel(x, W1, b1, W2, b2):
    b1r = b1.reshape(1, _D_HID)
    b2r = b2.reshape(1, _N_EXPERTS)
    grid = (_TOKENS // _BLOCK,)
    logits = pl.pallas_call(
        _mlp_body,
        grid=grid,
        in_specs=[
            pl.BlockSpec((_BLOCK, _D_IN), lambda i: (i, 0)),
            pl.BlockSpec((_D_IN, _D_HID), lambda i: (0, 0)),
            pl.BlockSpec((1, _D_HID), lambda i: (0, 0)),
            pl.BlockSpec((_D_HID, _N_EXPERTS), lambda i: (0, 0)),
            pl.BlockSpec((1, _N_EXPERTS), lambda i: (0, 0)),
        ],
        out_specs=pl.BlockSpec((_BLOCK, _N_EXPERTS), lambda i: (i, 0)),
        out_shape=jax.ShapeDtypeStruct((_TOKENS, _N_EXPERTS), jnp.float32),
        compiler_params=pltpu.CompilerParams(
            dimension_semantics=("parallel",)),
    )(x, W1, b1r, W2, b2r)
    i1, i2, g1, g2 = _sc_topk(logits.reshape(_TOKENS * _N_EXPERTS))
    idx = jnp.stack([i1, i2], axis=-1)
    gates = jnp.stack([g1, g2], axis=-1)
    return idx, gates


# speedup vs baseline: 1.1339x; 1.1339x over previous
import functools

import jax
import jax.numpy as jnp
from jax import lax
from jax.experimental import pallas as pl
from jax.experimental.pallas import tpu as pltpu
from jax.experimental.pallas import tpu_sc as plsc

_TOKENS = 32768
_D_IN = 768
_D_HID = 256
_N_EXPERTS = 64
_BLOCK = 4096

_HALF = _TOKENS // 2
_NW = 32
_TPW = _HALF // _NW
_LANES = 16
_CHUNKS = _TPW // _LANES


def _mlp_body(x_ref, w1_ref, b1_ref, w2_ref, b2_ref, logits_ref):
    h = jnp.dot(x_ref[...], w1_ref[...], preferred_element_type=jnp.float32)
    h = jnp.maximum(h + b1_ref[...], 0.0)
    logits = jnp.dot(h, w2_ref[...], preferred_element_type=jnp.float32)
    logits_ref[...] = logits + b2_ref[...]


def _sc_topk_body(logits_hbm, i1_hbm, i2_hbm, g1_hbm, g2_hbm,
                  lv, i1v, i2v, g1v, g2v):
    wid = lax.axis_index("s") * 2 + lax.axis_index("c")
    base = wid * _TPW
    pltpu.sync_copy(logits_hbm.at[pl.ds(base * _N_EXPERTS,
                                        _TPW * _N_EXPERTS)], lv)

    lane = lax.broadcasted_iota(jnp.int32, (_LANES,), 0)

    def chunk(c, _):
        flat0 = (c * _LANES + lane) * _N_EXPERTS
        m1 = plsc.load_gather(lv, [flat0])
        i1 = jnp.zeros((_LANES,), jnp.int32)
        m2 = jnp.full((_LANES,), -jnp.inf, jnp.float32)
        i2 = jnp.zeros((_LANES,), jnp.int32)
        for e in range(1, _N_EXPERTS):
            v = plsc.load_gather(lv, [flat0 + e])
            gt1 = v > m1
            gt2 = v > m2
            m2 = jnp.where(gt1, m1, jnp.where(gt2, v, m2))
            i2 = jnp.where(gt1, i1, jnp.where(gt2, e, i2))
            m1 = jnp.where(gt1, v, m1)
            i1 = jnp.where(gt1, e, i1)
        ex = jnp.exp(m2 - m1)
        denom = 1.0 + ex
        sl = pl.ds(c * _LANES, _LANES)
        i1v[sl] = i1
        i2v[sl] = i2
        g1v[sl] = 1.0 / denom
        g2v[sl] = ex / denom
        return _

    lax.fori_loop(0, _CHUNKS, chunk, 0)
    pltpu.sync_copy(i1v, i1_hbm.at[pl.ds(base, _TPW)])
    pltpu.sync_copy(i2v, i2_hbm.at[pl.ds(base, _TPW)])
    pltpu.sync_copy(g1v, g1_hbm.at[pl.ds(base, _TPW)])
    pltpu.sync_copy(g2v, g2_hbm.at[pl.ds(base, _TPW)])


_sc_topk = functools.partial(
    pl.kernel,
    mesh=plsc.VectorSubcoreMesh(core_axis_name="c", subcore_axis_name="s"),
    out_type=[
        jax.ShapeDtypeStruct((_HALF,), jnp.int32),
        jax.ShapeDtypeStruct((_HALF,), jnp.int32),
        jax.ShapeDtypeStruct((_HALF,), jnp.float32),
        jax.ShapeDtypeStruct((_HALF,), jnp.float32),
    ],
    scratch_types=[
        pltpu.VMEM((_TPW * _N_EXPERTS,), jnp.float32),
        pltpu.VMEM((_TPW,), jnp.int32),
        pltpu.VMEM((_TPW,), jnp.int32),
        pltpu.VMEM((_TPW,), jnp.float32),
        pltpu.VMEM((_TPW,), jnp.float32),
    ],
    compiler_params=pltpu.CompilerParams(needs_layout_passes=False),
)(_sc_topk_body)


def _mlp_half(x, W1, b1r, W2, b2r, half):
    nblk = _HALF // _BLOCK
    return pl.pallas_call(
        _mlp_body,
        grid=(nblk,),
        in_specs=[
            pl.BlockSpec((_BLOCK, _D_IN), lambda i, h=half: (i + h * nblk, 0)),
            pl.BlockSpec((_D_IN, _D_HID), lambda i: (0, 0)),
            pl.BlockSpec((1, _D_HID), lambda i: (0, 0)),
            pl.BlockSpec((_D_HID, _N_EXPERTS), lambda i: (0, 0)),
            pl.BlockSpec((1, _N_EXPERTS), lambda i: (0, 0)),
        ],
        out_specs=pl.BlockSpec((_BLOCK, _N_EXPERTS), lambda i: (i, 0)),
        out_shape=jax.ShapeDtypeStruct((_HALF, _N_EXPERTS), jnp.float32),
        compiler_params=pltpu.CompilerParams(
            dimension_semantics=("parallel",)),
    )(x, W1, b1r, W2, b2r)


@jax.jit
def kernel(x, W1, b1, W2, b2):
    b1r = b1.reshape(1, _D_HID)
    b2r = b2.reshape(1, _N_EXPERTS)
    logits0 = _mlp_half(x, W1, b1r, W2, b2r, 0)
    logits1 = _mlp_half(x, W1, b1r, W2, b2r, 1)
    a0 = _sc_topk(logits0.reshape(_HALF * _N_EXPERTS))
    a1 = _sc_topk(logits1.reshape(_HALF * _N_EXPERTS))
    i1 = jnp.concatenate([a0[0], a1[0]])
    i2 = jnp.concatenate([a0[1], a1[1]])
    g1 = jnp.concatenate([a0[2], a1[2]])
    g2 = jnp.concatenate([a0[3], a1[3]])
    idx = jnp.stack([i1, i2], axis=-1)
    gates = jnp.stack([g1, g2], axis=-1)
    return idx, gates


# speedup vs baseline: 1.2325x; 1.0870x over previous
import functools

import jax
import jax.numpy as jnp
from jax.experimental import pallas as pl
from jax.experimental.pallas import tpu as pltpu

_TOKENS = 32768
_D_IN = 768
_D_HID = 256
_N_EXPERTS = 64
_BLOCK = 4096
_KSPLIT = 2
_KDIM = _D_IN // _KSPLIT


def _gating_body(x_ref, w1_ref, b1_ref, w2_ref, b2_ref, idx_ref, gate_ref,
                 h_acc):
    k = pl.program_id(1)
    part = jnp.dot(x_ref[...], w1_ref[...], preferred_element_type=jnp.float32)

    @pl.when(k == 0)
    def _():
        h_acc[...] = part

    @pl.when(k == _KSPLIT - 1)
    def _():
        h = jnp.maximum(h_acc[...] + part + b1_ref[...], 0.0)
        logits = jnp.dot(h, w2_ref[...], preferred_element_type=jnp.float32)
        logits = logits + b2_ref[...]

        neg_iota = -jax.lax.broadcasted_iota(
            jnp.int32, logits.shape, 1).astype(jnp.float32)
        ninf = jnp.float32(-jnp.inf)
        m1 = jnp.max(logits, axis=1, keepdims=True)
        ni1 = jnp.max(jnp.where(logits == m1, neg_iota, ninf), axis=1,
                      keepdims=True)
        masked = jnp.where(neg_iota == ni1, ninf, logits)
        m2 = jnp.max(masked, axis=1, keepdims=True)
        ni2 = jnp.max(jnp.where(masked == m2, neg_iota, ninf), axis=1,
                      keepdims=True)
        i1 = (-ni1).astype(jnp.int32)
        i2 = (-ni2).astype(jnp.int32)

        e = jnp.exp(m2 - m1)
        denom = 1.0 + e
        g1 = 1.0 / denom
        g2 = e / denom

        idx_ref[...] = jnp.concatenate([i1, i2], axis=1)
        gate_ref[...] = jnp.concatenate([g1, g2], axis=1)


@functools.partial(jax.jit, static_argnames=("interpret",))
def kernel(x, W1, b1, W2, b2, interpret=False):
    b1r = b1.reshape(1, _D_HID)
    b2r = b2.reshape(1, _N_EXPERTS)
    grid = (_TOKENS // _BLOCK, _KSPLIT)
    idx, gates = pl.pallas_call(
        _gating_body,
        grid=grid,
        in_specs=[
            pl.BlockSpec((_BLOCK, _KDIM), lambda i, k: (i, k)),
            pl.BlockSpec((_KDIM, _D_HID), lambda i, k: (k, 0)),
            pl.BlockSpec((1, _D_HID), lambda i, k: (0, 0)),
            pl.BlockSpec((_D_HID, _N_EXPERTS), lambda i, k: (0, 0)),
            pl.BlockSpec((1, _N_EXPERTS), lambda i, k: (0, 0)),
        ],
        out_specs=[
            pl.BlockSpec((_BLOCK, 2), lambda i, k: (i, 0)),
            pl.BlockSpec((_BLOCK, 2), lambda i, k: (i, 0)),
        ],
        out_shape=[
            jax.ShapeDtypeStruct((_TOKENS, 2), jnp.int32),
            jax.ShapeDtypeStruct((_TOKENS, 2), jnp.float32),
        ],
        scratch_shapes=[pltpu.VMEM((_BLOCK, _D_HID), jnp.float32)],
        compiler_params=pltpu.CompilerParams(
            dimension_semantics=("parallel", "arbitrary")),
        interpret=interpret,
    )(x, W1, b1r, W2, b2r)
    return idx, gates


# speedup vs baseline: 1.6169x; 1.3119x over previous
import functools

import jax
import jax.numpy as jnp
from jax.experimental import pallas as pl
from jax.experimental.pallas import tpu as pltpu

_TOKENS = 32768
_D_IN = 768
_D_HID = 256
_N_EXPERTS = 64
_BLOCK = 4096


def _gating_body(x_ref, w1_ref, b1_ref, w2_ref, b2_ref, idx_ref, gate_ref):
    h = jnp.dot(x_ref[...], w1_ref[...], preferred_element_type=jnp.float32)
    h = jnp.maximum(h + b1_ref[...], 0.0)
    logits = jnp.dot(h, w2_ref[...], preferred_element_type=jnp.float32)
    logits = logits + b2_ref[...]

    neg_iota = -jax.lax.broadcasted_iota(
        jnp.int32, logits.shape, 1).astype(jnp.float32)
    ninf = jnp.float32(-jnp.inf)
    m1 = jnp.max(logits, axis=1, keepdims=True)
    ni1 = jnp.max(jnp.where(logits == m1, neg_iota, ninf), axis=1,
                  keepdims=True)
    masked = jnp.where(neg_iota == ni1, ninf, logits)
    m2 = jnp.max(masked, axis=1, keepdims=True)
    ni2 = jnp.max(jnp.where(masked == m2, neg_iota, ninf), axis=1,
                  keepdims=True)
    i1 = (-ni1).astype(jnp.int32)
    i2 = (-ni2).astype(jnp.int32)

    e = jnp.exp(m2 - m1)
    denom = 1.0 + e
    g1 = 1.0 / denom
    g2 = e / denom

    idx_ref[...] = jnp.concatenate([i1, i2], axis=1)
    gate_ref[...] = jnp.concatenate([g1, g2], axis=1)


@functools.partial(jax.jit, static_argnames=("interpret",))
def kernel(x, W1, b1, W2, b2, interpret=False):
    b1r = b1.reshape(1, _D_HID)
    b2r = b2.reshape(1, _N_EXPERTS)
    grid = (_TOKENS // _BLOCK,)
    idx, gates = pl.pallas_call(
        _gating_body,
        grid=grid,
        in_specs=[
            pl.BlockSpec((_BLOCK, _D_IN), lambda i: (i, 0)),
            pl.BlockSpec((_D_IN, _D_HID), lambda i: (0, 0)),
            pl.BlockSpec((1, _D_HID), lambda i: (0, 0)),
            pl.BlockSpec((_D_HID, _N_EXPERTS), lambda i: (0, 0)),
            pl.BlockSpec((1, _N_EXPERTS), lambda i: (0, 0)),
        ],
        out_specs=[
            pl.BlockSpec((_BLOCK, 2), lambda i: (i, 0)),
            pl.BlockSpec((_BLOCK, 2), lambda i: (i, 0)),
        ],
        out_shape=[
            jax.ShapeDtypeStruct((_TOKENS, 2), jnp.int32),
            jax.ShapeDtypeStruct((_TOKENS, 2), jnp.float32),
        ],
        compiler_params=pltpu.CompilerParams(
            dimension_semantics=("parallel",)),
        interpret=interpret,
    )(x, W1, b1r, W2, b2r)
    return idx, gates


# speedup vs baseline: 1.6199x; 1.0018x over previous
import functools

import jax
import jax.numpy as jnp
from jax.experimental import pallas as pl
from jax.experimental.pallas import tpu as pltpu

_TOKENS = 32768
_D_IN = 768
_D_HID = 256
_N_EXPERTS = 64
_BLOCK = 4096
_HB = _BLOCK // 2


def _gating_body(xa_ref, xb_ref, w1_ref, b1_ref, w2_ref, b2_ref,
                 idx_ref, gate_ref):
    h_a = jnp.dot(xa_ref[...], w1_ref[...], preferred_element_type=jnp.float32)
    h_b = jnp.dot(xb_ref[...], w1_ref[...], preferred_element_type=jnp.float32)
    h = jnp.concatenate([h_a, h_b], axis=0)
    h = jnp.maximum(h + b1_ref[...], 0.0)
    logits = jnp.dot(h, w2_ref[...], preferred_element_type=jnp.float32)
    logits = logits + b2_ref[...]

    neg_iota = -jax.lax.broadcasted_iota(
        jnp.int32, logits.shape, 1).astype(jnp.float32)
    ninf = jnp.float32(-jnp.inf)
    m1 = jnp.max(logits, axis=1, keepdims=True)
    ni1 = jnp.max(jnp.where(logits == m1, neg_iota, ninf), axis=1,
                  keepdims=True)
    masked = jnp.where(neg_iota == ni1, ninf, logits)
    m2 = jnp.max(masked, axis=1, keepdims=True)
    ni2 = jnp.max(jnp.where(masked == m2, neg_iota, ninf), axis=1,
                  keepdims=True)
    i1 = (-ni1).astype(jnp.int32)
    i2 = (-ni2).astype(jnp.int32)

    e = jnp.exp(m2 - m1)
    denom = 1.0 + e
    g1 = 1.0 / denom
    g2 = e / denom

    idx_ref[...] = jnp.concatenate([i1, i2], axis=1)
    gate_ref[...] = jnp.concatenate([g1, g2], axis=1)


@functools.partial(jax.jit, static_argnames=("interpret",))
def kernel(x, W1, b1, W2, b2, interpret=False):
    b1r = b1.reshape(1, _D_HID)
    b2r = b2.reshape(1, _N_EXPERTS)
    grid = (_TOKENS // _BLOCK,)
    idx, gates = pl.pallas_call(
        _gating_body,
        grid=grid,
        in_specs=[
            pl.BlockSpec((_HB, _D_IN), lambda i: (2 * i, 0)),
            pl.BlockSpec((_HB, _D_IN), lambda i: (2 * i + 1, 0)),
            pl.BlockSpec((_D_IN, _D_HID), lambda i: (0, 0)),
            pl.BlockSpec((1, _D_HID), lambda i: (0, 0)),
            pl.BlockSpec((_D_HID, _N_EXPERTS), lambda i: (0, 0)),
            pl.BlockSpec((1, _N_EXPERTS), lambda i: (0, 0)),
        ],
        out_specs=[
            pl.BlockSpec((_BLOCK, 2), lambda i: (i, 0)),
            pl.BlockSpec((_BLOCK, 2), lambda i: (i, 0)),
        ],
        out_shape=[
            jax.ShapeDtypeStruct((_TOKENS, 2), jnp.int32),
            jax.ShapeDtypeStruct((_TOKENS, 2), jnp.float32),
        ],
        compiler_params=pltpu.CompilerParams(
            dimension_semantics=("parallel",)),
        interpret=interpret,
    )(x, x, W1, b1r, W2, b2r)
    return idx, gates
